# Initial kernel scaffold; baseline (speedup 1.0000x reference)
import jax
import jax.numpy as jnp
from jax.experimental import pallas as pl


def kernel(x_user, x_item, edge_index_u2i, edge_index_i2u, Wl0_u2i, bl0_u2i, Wr0_u2i, Wl0_i2u, bl0_i2u, Wr0_i2u, Wl1_u2i, bl1_u2i, Wr1_u2i, Wl1_i2u, bl1_i2u, Wr1_i2u):
    raise NotImplementedError("write your pallas kernel here")


# speedup vs baseline: 1.0474x; 1.0474x over previous
import functools

import jax
import jax.numpy as jnp
from jax import lax
from jax.experimental import pallas as pl
from jax.experimental.pallas import tpu as pltpu
from jax.experimental.pallas import tpu_sc as plsc

N = 50000
D = 128
H = 128
O = 64
E = 300000

NC = 2
NS = 16
LANE = 16
CH = 128

EPT = 18816
EPAD = EPT * NS
EB = 2688
NBLK = EPT // EB
CAP = EB + CH

NRANGE = 4
RW = 12544
ACC_R = RW + 16
GLOC = RW
ZR_T = RW // NS
NOUT_R = NRANGE * RW
GARBAGE = N


def _sc_agg_body(with_cnt, *refs):
    (t0, t1, s0, d0, s1, d1) = refs[:6]
    n_out = 4 if with_cnt else 2
    outs = refs[6:6 + n_out]
    (sblk, dblk, csrc, cdst, stage_d, gbuf, sem) = refs[6 + n_out:6 + n_out + 7]
    acc = refs[6 + n_out + 7]
    tbls = (t0, t1)
    srcs = (s0, s1)
    dsts = (d0, d1)

    c = lax.axis_index("c")
    s = lax.axis_index("s")

    zero16i = jnp.zeros((LANE,), jnp.int32)
    gloc16 = jnp.full((LANE,), GLOC, jnp.int32)
    iota16 = lax.iota(jnp.int32, LANE)

    def fill_gbuf(val):
        v16 = jnp.full((LANE,), val, jnp.float32)

        def fill(r, carry):
            for j in range(D // LANE):
                gbuf[r, pl.ds(j * LANE, LANE)] = v16
            return carry

        lax.fori_loop(0, CH, fill, 0)

    def zero_acc():
        for kz in range(ZR_T // CH):
            pltpu.sync_copy(gbuf, acc.at[pl.ds(s * ZR_T + kz * CH, CH)])
        rem = ZR_T - (ZR_T // CH) * CH
        if rem:
            pltpu.sync_copy(gbuf.at[pl.ds(0, rem)],
                            acc.at[pl.ds(s * ZR_T + (ZR_T // CH) * CH, rem)])
        plsc.subcore_barrier()

    def copy_out(out_ref, rng):
        plsc.subcore_barrier()
        pltpu.sync_copy(acc.at[pl.ds(s * ZR_T, ZR_T)],
                        out_ref.at[pl.ds(rng * RW + s * ZR_T, ZR_T)])
        plsc.subcore_barrier()

    def compact_block(lo):
        def step(i, cur):
            sl = pl.ds(i * LANE, LANE)
            d16 = dblk[sl]
            s16 = sblk[sl]
            m = (d16 >= lo) & (d16 < lo + RW)
            mi = m.astype(jnp.int32)
            offs = cur + plsc.cumsum(mi) - mi
            plsc.store_scatter(csrc, [offs], s16, mask=m)
            plsc.store_scatter(cdst, [offs], d16 - lo, mask=m)
            return cur + jnp.sum(mi)

        k = lax.fori_loop(0, EB // LANE, step, 0)
        for j in range(CH // LANE):
            idx = k + j * LANE + iota16
            plsc.store_scatter(csrc, [idx], zero16i)
            plsc.store_scatter(cdst, [idx], gloc16)
        return (k + CH - 1) // CH

    def stage_chunk(g):
        base = g * CH
        for j in range(CH // LANE):
            stage_d[pl.ds(j * LANE, LANE)] = cdst[pl.ds(base + j * LANE, LANE)]

    def run_pass(rel, lo, out_ref, rng, is_cnt):
        fill_gbuf(0.0)
        zero_acc()
        if is_cnt:
            fill_gbuf(1.0)

        for blk in range(NBLK):
            ebase = s * EPT + blk * EB
            pltpu.sync_copy(srcs[rel].at[pl.ds(ebase, EB)], sblk)
            pltpu.sync_copy(dsts[rel].at[pl.ds(ebase, EB)], dblk)
            nch = compact_block(lo)

            if is_cnt:
                def chunk(g, carry):
                    stage_chunk(g)
                    pltpu.sync_copy(gbuf, acc.at[stage_d], add=True)
                    return carry
            else:
                def chunk(g, carry):
                    stage_chunk(g)
                    pltpu.async_copy(tbls[rel].at[csrc.at[pl.ds(g * CH, CH)]],
                                     gbuf, sem).wait()
                    pltpu.sync_copy(gbuf, acc.at[stage_d], add=True)
                    return carry

            lax.fori_loop(0, nch, chunk, 0)
        copy_out(out_ref, rng)

    for rel in range(2):
        for p in range(2):
            rng = 2 * c + p
            lo = rng * RW
            run_pass(rel, lo, outs[rel], rng, False)
            if with_cnt:
                run_pass(rel, lo, outs[2 + rel], rng, True)


def _make_sc_agg(with_cnt):
    n_out = 4 if with_cnt else 2
    out_type = [jax.ShapeDtypeStruct((NOUT_R, D), jnp.float32)] * n_out
    mesh = plsc.VectorSubcoreMesh(core_axis_name="c", subcore_axis_name="s")
    return pl.kernel(
        functools.partial(_sc_agg_body, with_cnt),
        out_type=out_type,
        mesh=mesh,
        scratch_types=[
            pltpu.VMEM((EB,), jnp.int32),
            pltpu.VMEM((EB,), jnp.int32),
            pltpu.VMEM((CAP,), jnp.int32),
            pltpu.VMEM((CAP,), jnp.int32),
            pltpu.VMEM((CH,), jnp.int32),
            pltpu.VMEM((CH, D), jnp.float32),
            pltpu.SemaphoreType.DMA,
            pltpu.VMEM_SHARED((ACC_R, D), jnp.float32),
        ],
        compiler_params=pltpu.CompilerParams(needs_layout_passes=False),
        name="sc_agg_cnt" if with_cnt else "sc_agg",
    )


_sc_agg_l0 = _make_sc_agg(with_cnt=True)
_sc_agg_l1 = _make_sc_agg(with_cnt=False)


def _make_dense(dout, relu):
    BR = 400
    grid = (N // BR,)

    def body(agg_ref, cnt_ref, xd_ref, wl_ref, bl_ref, wr_ref, o_ref):
        mean = agg_ref[...] / jnp.maximum(cnt_ref[...], 1.0)
        acc = jnp.dot(xd_ref[...], wr_ref[...],
                      preferred_element_type=jnp.float32)
        acc = acc + jnp.dot(mean, wl_ref[...],
                            preferred_element_type=jnp.float32)
        r = acc + bl_ref[...]
        if relu:
            r = jnp.maximum(r, 0.0)
        o_ref[...] = r

    blk = lambda i: (i, 0)
    fix = lambda i: (0, 0)
    return pl.pallas_call(
        body,
        grid=grid,
        in_specs=[
            pl.BlockSpec((BR, D), blk),
            pl.BlockSpec((BR, D), blk),
            pl.BlockSpec((BR, D), blk),
            pl.BlockSpec((D, dout), fix),
            pl.BlockSpec((1, dout), fix),
            pl.BlockSpec((D, dout), fix),
        ],
        out_specs=pl.BlockSpec((BR, dout), blk),
        out_shape=jax.ShapeDtypeStruct((N, dout), jnp.float32),
    )


_dense_l0 = _make_dense(H, relu=True)
_dense_l1 = _make_dense(O, relu=False)


def _pad_edges(ei):
    src = ei[0]
    dst = ei[1]
    pad = EPAD - E
    src_p = jnp.concatenate([src, jnp.zeros((pad,), jnp.int32)])
    dst_p = jnp.concatenate([dst, jnp.full((pad,), GARBAGE, jnp.int32)])
    return src_p, dst_p


def kernel(x_user, x_item, edge_index_u2i, edge_index_i2u,
           Wl0_u2i, bl0_u2i, Wr0_u2i, Wl0_i2u, bl0_i2u, Wr0_i2u,
           Wl1_u2i, bl1_u2i, Wr1_u2i, Wl1_i2u, bl1_i2u, Wr1_i2u):
    srcu, dstu = _pad_edges(edge_index_u2i)
    srci, dsti = _pad_edges(edge_index_i2u)
    aggu, aggi, cntu, cnti = _sc_agg_l0(x_user, x_item, srcu, dstu,
                                        srci, dsti)

    item1 = _dense_l0(aggu, cntu, x_item,
                      Wl0_u2i, bl0_u2i.reshape(1, H), Wr0_u2i)
    user1 = _dense_l0(aggi, cnti, x_user,
                      Wl0_i2u, bl0_i2u.reshape(1, H), Wr0_i2u)

    aggu2, aggi2 = _sc_agg_l1(user1, item1, srcu, dstu, srci, dsti)

    item2 = _dense_l1(aggu2, cntu, item1,
                      Wl1_u2i, bl1_u2i.reshape(1, O), Wr1_u2i)
    user2 = _dense_l1(aggi2, cnti, user1,
                      Wl1_i2u, bl1_i2u.reshape(1, O), Wr1_i2u)
    return (user2, item2)
